# baseline (speedup 1.0000x reference)
import functools

import jax
import jax.numpy as jnp
from jax import lax
from jax.experimental import pallas as pl
from jax.experimental.pallas import tpu as pltpu
from jax.experimental.pallas import tpu_sc as plsc

B, L, D = 4096, 50, 64
NC, NS, LANES = 2, 16, 16
NW = NC * NS
BPW = B // NW
KJ = 4
ROWS = KJ * BPW
NB = 3
_ROUNDS = [(r * ROWS, ROWS) for r in range(12)] + [(12 * ROWS, 2 * BPW)]

_mesh = plsc.VectorSubcoreMesh(core_axis_name="c", subcore_axis_name="s")


@functools.partial(
    pl.kernel,
    out_type=jax.ShapeDtypeStruct((B, D), jnp.float32),
    mesh=_mesh,
    scratch_types=[
        pltpu.VMEM((L * BPW,), jnp.int32),
        pltpu.VMEM((ROWS, D), jnp.float32),
        pltpu.VMEM((ROWS, D), jnp.float32),
        pltpu.VMEM((ROWS, D), jnp.float32),
        pltpu.VMEM((ROWS,), jnp.int32),
        pltpu.VMEM((2 * BPW,), jnp.int32),
        pltpu.VMEM_SHARED((NS * BPW, D), jnp.float32),
        pltpu.SemaphoreType.DMA,
        pltpu.SemaphoreType.DMA,
        pltpu.SemaphoreType.DMA,
        pltpu.SemaphoreType.DMA,
        pltpu.SemaphoreType.DMA,
        pltpu.SemaphoreType.DMA,
    ],
    compiler_params=pltpu.CompilerParams(use_tc_tiling_on_sc=False),
)
def _feature_sc(xw_hbm, f_hbm, out_hbm, idx_v, buf_0, buf_1, buf_2,
                dst_v, dtl_v, acc_sh, sg_0, sg_1, sg_2, ss_0, ss_1, ss_2):
    c = lax.axis_index("c")
    s = lax.axis_index("s")
    wid = c * NS + s
    base = wid * BPW
    region = s * BPW

    pltpu.sync_copy(xw_hbm.at[pl.ds(wid * (L * BPW), L * BPW)], idx_v)

    for t in range(ROWS // LANES):
        col = (t * LANES) % BPW
        v16 = lax.iota(jnp.int32, LANES) + (region + col)
        dst_v[pl.ds(t * LANES, LANES)] = v16
        if t < (2 * BPW) // LANES:
            dtl_v[pl.ds(t * LANES, LANES)] = v16

    bufs = (buf_0, buf_1, buf_2)
    sg = (sg_0, sg_1, sg_2)
    ss = (ss_0, ss_1, ss_2)
    gathers = {}
    scatters = {}

    for r in range(2):
        off, n = _ROUNDS[r]
        gathers[r] = pltpu.async_copy(
            f_hbm.at[idx_v.at[pl.ds(off, n)]], bufs[r].at[pl.ds(0, n)], sg[r])

    def _zrow(i, carry):
        for k in range(D // LANES):
            buf_2[i, pl.ds(k * LANES, LANES)] = jnp.zeros((LANES,), jnp.float32)
        return carry

    lax.fori_loop(0, BPW, _zrow, 0)
    pltpu.sync_copy(buf_2.at[pl.ds(0, BPW)], acc_sh.at[pl.ds(region, BPW)])
    off, n = _ROUNDS[2]
    gathers[2] = pltpu.async_copy(
        f_hbm.at[idx_v.at[pl.ds(off, n)]], bufs[2].at[pl.ds(0, n)], sg[2])

    NT = len(_ROUNDS)
    for r in range(NT):
        off, n = _ROUNDS[r]
        pb = r % NB
        gathers[r].wait()
        dref = dst_v if n == ROWS else dtl_v
        scatters[r] = pltpu.async_copy(
            bufs[pb].at[pl.ds(0, n)], acc_sh.at[dref], ss[pb], add=True)
        nxt = r + NB
        if nxt < NT:
            scatters[r].wait()
            off2, n2 = _ROUNDS[nxt]
            gathers[nxt] = pltpu.async_copy(
                f_hbm.at[idx_v.at[pl.ds(off2, n2)]],
                bufs[pb].at[pl.ds(0, n2)], sg[pb])
    for r in range(NT - NB, NT):
        scatters[r].wait()

    pltpu.sync_copy(acc_sh.at[pl.ds(region, BPW)], out_hbm.at[pl.ds(base, BPW)])


_TBLK = 512


def _t_body(ft_ref, o_ref):
    blk = ft_ref[...]
    m_i = lax.broadcasted_iota(jnp.int32, (_TBLK // 2, _TBLK), 0)
    j_i = lax.broadcasted_iota(jnp.int32, (_TBLK // 2, _TBLK), 1)
    sel_e = (j_i == 2 * m_i).astype(jnp.float32)
    sel_o = (j_i == 2 * m_i + 1).astype(jnp.float32)
    dn = (((1,), (1,)), ((), ()))
    o_ref[:, 0:D] = lax.dot_general(sel_e, blk, dn)
    o_ref[:, D:2 * D] = lax.dot_general(sel_o, blk, dn)


_transpose_tc = pl.pallas_call(
    _t_body,
    grid=((100000 + _TBLK - 1) // _TBLK,),
    in_specs=[pl.BlockSpec((D, _TBLK), lambda i: (0, i))],
    out_specs=pl.BlockSpec((_TBLK // 2, 2 * D), lambda i: (i, 0)),
    out_shape=jax.ShapeDtypeStruct((50000, 2 * D), jnp.float32),
)


def kernel(x, F):
    xw = (x.astype(jnp.int32)
          .reshape(NW, BPW, L)
          .transpose(0, 2, 1)
          .reshape(NW * L * BPW))
    f_lin = _transpose_tc(jnp.transpose(F)).reshape(100000, D)
    return _feature_sc(xw, f_lin)


# speedup vs baseline: 1.6415x; 1.6415x over previous
import functools

import jax
import jax.numpy as jnp
from jax import lax
from jax.experimental import pallas as pl
from jax.experimental.pallas import tpu as pltpu
from jax.experimental.pallas import tpu_sc as plsc

B, L, D = 4096, 50, 64
NC, NS, LANES = 2, 16, 16
NW = NC * NS
BPW = B // NW
KJ = 3
ROWS = KJ * BPW
NB = 4
_ROUNDS = [(r * ROWS, ROWS) for r in range(16)] + [(16 * ROWS, 2 * BPW)]

_mesh = plsc.VectorSubcoreMesh(core_axis_name="c", subcore_axis_name="s")


@functools.partial(
    pl.kernel,
    out_type=jax.ShapeDtypeStruct((B, D), jnp.float32),
    mesh=_mesh,
    scratch_types=[
        pltpu.VMEM((L * BPW,), jnp.int32),
        pltpu.VMEM((ROWS, D), jnp.float32),
        pltpu.VMEM((ROWS, D), jnp.float32),
        pltpu.VMEM((ROWS, D), jnp.float32),
        pltpu.VMEM((ROWS, D), jnp.float32),
        pltpu.VMEM((ROWS,), jnp.int32),
        pltpu.VMEM((2 * BPW,), jnp.int32),
        pltpu.VMEM_SHARED((NS * BPW, D), jnp.float32),
        pltpu.SemaphoreType.DMA,
        pltpu.SemaphoreType.DMA,
        pltpu.SemaphoreType.DMA,
        pltpu.SemaphoreType.DMA,
        pltpu.SemaphoreType.DMA,
        pltpu.SemaphoreType.DMA,
        pltpu.SemaphoreType.DMA,
        pltpu.SemaphoreType.DMA,
    ],
    compiler_params=pltpu.CompilerParams(use_tc_tiling_on_sc=False),
)
def _feature_sc(xw_hbm, f_hbm, out_hbm, idx_v, buf_0, buf_1, buf_2, buf_3,
                dst_v, dtl_v, acc_sh, sg_0, sg_1, sg_2, sg_3,
                ss_0, ss_1, ss_2, ss_3):
    c = lax.axis_index("c")
    s = lax.axis_index("s")
    wid = c * NS + s
    base = wid * BPW
    region = s * BPW

    pltpu.sync_copy(xw_hbm.at[pl.ds(wid * (L * BPW), L * BPW)], idx_v)

    for t in range(ROWS // LANES):
        col = (t * LANES) % BPW
        v16 = lax.iota(jnp.int32, LANES) + (region + col)
        dst_v[pl.ds(t * LANES, LANES)] = v16
        if t < (2 * BPW) // LANES:
            dtl_v[pl.ds(t * LANES, LANES)] = v16

    bufs = (buf_0, buf_1, buf_2, buf_3)
    sg = (sg_0, sg_1, sg_2, sg_3)
    ss = (ss_0, ss_1, ss_2, ss_3)
    gathers = {}
    scatters = {}

    for r in range(NB - 1):
        off, n = _ROUNDS[r]
        gathers[r] = pltpu.async_copy(
            f_hbm.at[idx_v.at[pl.ds(off, n)]], bufs[r].at[pl.ds(0, n)], sg[r])

    def _zrow(i, carry):
        for k in range(D // LANES):
            buf_3[i, pl.ds(k * LANES, LANES)] = jnp.zeros((LANES,), jnp.float32)
        return carry

    lax.fori_loop(0, BPW, _zrow, 0)
    pltpu.sync_copy(buf_3.at[pl.ds(0, BPW)], acc_sh.at[pl.ds(region, BPW)])
    off, n = _ROUNDS[NB - 1]
    gathers[NB - 1] = pltpu.async_copy(
        f_hbm.at[idx_v.at[pl.ds(off, n)]], bufs[NB - 1].at[pl.ds(0, n)],
        sg[NB - 1])

    NT = len(_ROUNDS)
    for r in range(NT):
        off, n = _ROUNDS[r]
        pb = r % NB
        gathers[r].wait()
        dref = dst_v if n == ROWS else dtl_v
        scatters[r] = pltpu.async_copy(
            bufs[pb].at[pl.ds(0, n)], acc_sh.at[dref], ss[pb], add=True)
        nxt = r + NB
        if nxt < NT:
            scatters[r].wait()
            off2, n2 = _ROUNDS[nxt]
            gathers[nxt] = pltpu.async_copy(
                f_hbm.at[idx_v.at[pl.ds(off2, n2)]],
                bufs[pb].at[pl.ds(0, n2)], sg[pb])
    for r in range(NT - NB, NT):
        scatters[r].wait()

    pltpu.sync_copy(acc_sh.at[pl.ds(region, BPW)], out_hbm.at[pl.ds(base, BPW)])


def kernel(x, F):
    xw = (x.astype(jnp.int32)
          .reshape(NW, BPW, L)
          .transpose(0, 2, 1)
          .reshape(NW * L * BPW))
    return _feature_sc(xw, F)


# speedup vs baseline: 1.6508x; 1.0056x over previous
import functools

import jax
import jax.numpy as jnp
from jax import lax
from jax.experimental import pallas as pl
from jax.experimental.pallas import tpu as pltpu
from jax.experimental.pallas import tpu_sc as plsc

B, L, D = 4096, 50, 64
NC, NS, LANES = 2, 16, 16
NW = NC * NS
BPW = B // NW
KJ = 4
ROWS = KJ * BPW
NB = 3
_ROUNDS = [(r * ROWS, ROWS) for r in range(12)] + [(12 * ROWS, 2 * BPW)]

_mesh = plsc.VectorSubcoreMesh(core_axis_name="c", subcore_axis_name="s")


@functools.partial(
    pl.kernel,
    out_type=jax.ShapeDtypeStruct((B, D), jnp.float32),
    mesh=_mesh,
    scratch_types=[
        pltpu.VMEM((L * BPW,), jnp.int32),
        pltpu.VMEM((ROWS, D), jnp.float32),
        pltpu.VMEM((ROWS, D), jnp.float32),
        pltpu.VMEM((ROWS, D), jnp.float32),
        pltpu.VMEM((ROWS,), jnp.int32),
        pltpu.VMEM((2 * BPW,), jnp.int32),
        pltpu.VMEM_SHARED((NS * BPW, D), jnp.float32),
        pltpu.SemaphoreType.DMA,
        pltpu.SemaphoreType.DMA,
        pltpu.SemaphoreType.DMA,
        pltpu.SemaphoreType.DMA,
        pltpu.SemaphoreType.DMA,
        pltpu.SemaphoreType.DMA,
    ],
    compiler_params=pltpu.CompilerParams(use_tc_tiling_on_sc=False),
)
def _feature_sc(xw_hbm, f_hbm, out_hbm, idx_v, buf_0, buf_1, buf_2,
                dst_v, dtl_v, acc_sh, sg_0, sg_1, sg_2, ss_0, ss_1, ss_2):
    c = lax.axis_index("c")
    s = lax.axis_index("s")
    wid = c * NS + s
    base = wid * BPW
    region = s * BPW

    pltpu.sync_copy(xw_hbm.at[pl.ds(wid * (L * BPW), L * BPW)], idx_v)

    for t in range(ROWS // LANES):
        col = (t * LANES) % BPW
        v16 = lax.iota(jnp.int32, LANES) + (region + col)
        dst_v[pl.ds(t * LANES, LANES)] = v16
        if t < (2 * BPW) // LANES:
            dtl_v[pl.ds(t * LANES, LANES)] = v16

    bufs = (buf_0, buf_1, buf_2)
    sg = (sg_0, sg_1, sg_2)
    ss = (ss_0, ss_1, ss_2)
    gathers = {}
    scatters = {}

    for r in range(2):
        off, n = _ROUNDS[r]
        gathers[r] = pltpu.async_copy(
            f_hbm.at[idx_v.at[pl.ds(off, n)]], bufs[r].at[pl.ds(0, n)], sg[r])

    def _zrow(i, carry):
        for k in range(D // LANES):
            buf_2[i, pl.ds(k * LANES, LANES)] = jnp.zeros((LANES,), jnp.float32)
        return carry

    lax.fori_loop(0, BPW, _zrow, 0)
    pltpu.sync_copy(buf_2.at[pl.ds(0, BPW)], acc_sh.at[pl.ds(region, BPW)])
    off, n = _ROUNDS[2]
    gathers[2] = pltpu.async_copy(
        f_hbm.at[idx_v.at[pl.ds(off, n)]], bufs[2].at[pl.ds(0, n)], sg[2])

    NT = len(_ROUNDS)
    for r in range(NT):
        off, n = _ROUNDS[r]
        pb = r % NB
        gathers[r].wait()
        dref = dst_v if n == ROWS else dtl_v
        scatters[r] = pltpu.async_copy(
            bufs[pb].at[pl.ds(0, n)], acc_sh.at[dref], ss[pb], add=True)
        nxt = r + NB
        if nxt < NT:
            scatters[r].wait()
            off2, n2 = _ROUNDS[nxt]
            gathers[nxt] = pltpu.async_copy(
                f_hbm.at[idx_v.at[pl.ds(off2, n2)]],
                bufs[pb].at[pl.ds(0, n2)], sg[pb])
    for r in range(NT - NB, NT):
        scatters[r].wait()

    pltpu.sync_copy(acc_sh.at[pl.ds(region, BPW)], out_hbm.at[pl.ds(base, BPW)])


def kernel(x, F):
    xw = (x.astype(jnp.int32)
          .reshape(NW, BPW, L)
          .transpose(0, 2, 1)
          .reshape(NW * L * BPW))
    return _feature_sc(xw, F)
